# baseline (speedup 1.0000x reference)
import jax
import jax.numpy as jnp
from jax import lax
from jax.experimental import pallas as pl
from jax.experimental.pallas import tpu as pltpu

NSTRIPE = 8
EPS = 1e-06


def _body(x_ref, o_ref):
    X3 = x_ref[0]
    thw, c = X3.shape
    t = 8
    ng = t * NSTRIPE
    X4 = X3.reshape(ng, 16, c)
    A = X4[:, 0:8, :] + X4[:, 8:16, :]
    F = jnp.sum(A, axis=1) * (1.0 / 16.0)

    colv = jnp.sum(F * F, axis=1, keepdims=True)
    S8 = jnp.concatenate([colv[8 * k:8 * (k + 1)] for k in range(t)], axis=1)
    S8 = jnp.maximum(S8, EPS)

    tio = lax.broadcasted_iota(jnp.int32, (NSTRIPE, t), 1)
    m1 = jnp.max(S8, axis=1, keepdims=True)
    i1 = jnp.min(jnp.where(S8 == m1, tio, t), axis=1, keepdims=True)
    Sm = jnp.where(tio == i1, -1.0, S8)
    m2 = jnp.max(Sm, axis=1, keepdims=True)
    i2 = jnp.min(jnp.where(Sm == m2, tio, t), axis=1, keepdims=True)

    i1t = jnp.tile(i1, (t, 1))
    i2t = jnp.tile(i2, (t, 1))
    gio = lax.broadcasted_iota(jnp.int32, (ng, 1), 0)
    tg = gio // NSTRIPE
    Wg = jnp.where((tg == i1t) | (tg == i2t), 0.5, 0.0)

    FW = F * Wg
    o_ref[0] = jnp.sum(FW.reshape(t, NSTRIPE, c), axis=0)


def kernel(x):
    n, c, t, h, w = x.shape
    xt = x.transpose(0, 2, 3, 4, 1).reshape(n, t * h * w, c)
    out = pl.pallas_call(
        _body,
        grid=(n,),
        in_specs=[pl.BlockSpec((1, t * h * w, c), lambda i: (i, 0, 0))],
        out_specs=pl.BlockSpec((1, NSTRIPE, c), lambda i: (i, 0, 0)),
        out_shape=jax.ShapeDtypeStruct((n, NSTRIPE, c), jnp.float32),
        compiler_params=pltpu.CompilerParams(
            dimension_semantics=("parallel",)),
    )(xt)
    return out.reshape(n, NSTRIPE * c)


# speedup vs baseline: 1.0218x; 1.0218x over previous
import jax
import jax.numpy as jnp
from jax import lax
from jax.experimental import pallas as pl
from jax.experimental.pallas import tpu as pltpu

NSTRIPE = 8
EPS = 1e-06


def _body(x0_ref, x1_ref, o_ref):
    t = 8
    ng = t * NSTRIPE
    Fs = []
    colv = None
    for x_ref in (x0_ref, x1_ref):
        X3 = x_ref[0]
        thw, ch = X3.shape
        X4 = X3.reshape(ng, 16, ch)
        A = X4[:, 0:8, :] + X4[:, 8:16, :]
        F = jnp.sum(A, axis=1) * (1.0 / 16.0)
        Fs.append(F)
        cv = jnp.sum(F * F, axis=1, keepdims=True)
        colv = cv if colv is None else colv + cv
    S8 = jnp.concatenate([colv[8 * k:8 * (k + 1)] for k in range(t)], axis=1)
    S8 = jnp.maximum(S8, EPS)

    tio = lax.broadcasted_iota(jnp.int32, (NSTRIPE, t), 1)
    m1 = jnp.max(S8, axis=1, keepdims=True)
    i1 = jnp.min(jnp.where(S8 == m1, tio, t), axis=1, keepdims=True)
    Sm = jnp.where(tio == i1, -1.0, S8)
    m2 = jnp.max(Sm, axis=1, keepdims=True)
    i2 = jnp.min(jnp.where(Sm == m2, tio, t), axis=1, keepdims=True)

    i1t = jnp.tile(i1, (t, 1))
    i2t = jnp.tile(i2, (t, 1))
    gio = lax.broadcasted_iota(jnp.int32, (ng, 1), 0)
    tg = gio // NSTRIPE
    Wg = jnp.where((tg == i1t) | (tg == i2t), 0.5, 0.0)

    for j, F in enumerate(Fs):
        ch = F.shape[1]
        FW = F * Wg
        o_ref[0, :, j * ch:(j + 1) * ch] = jnp.sum(
            FW.reshape(t, NSTRIPE, ch), axis=0)


def kernel(x):
    n, c, t, h, w = x.shape
    xt = x.transpose(0, 2, 3, 4, 1).reshape(n, t * h * w, c)
    ch = c // 2
    out = pl.pallas_call(
        _body,
        grid=(n,),
        in_specs=[
            pl.BlockSpec((1, t * h * w, ch), lambda i: (i, 0, 0)),
            pl.BlockSpec((1, t * h * w, ch), lambda i: (i, 0, 1)),
        ],
        out_specs=pl.BlockSpec((1, NSTRIPE, c), lambda i: (i, 0, 0)),
        out_shape=jax.ShapeDtypeStruct((n, NSTRIPE, c), jnp.float32),
        compiler_params=pltpu.CompilerParams(
            dimension_semantics=("parallel",)),
    )(xt, xt)
    return out.reshape(n, NSTRIPE * c)


# speedup vs baseline: 1.1560x; 1.1313x over previous
import jax
import jax.numpy as jnp
from jax import lax
from jax.experimental import pallas as pl
from jax.experimental.pallas import tpu as pltpu

NSTRIPE = 8
EPS = 1e-06


def _body(x0_ref, x1_ref, o_ref):
    for b in range(x0_ref.shape[0]):
        _one(b, x0_ref, x1_ref, o_ref)


def _one(b, x0_ref, x1_ref, o_ref):
    t = 8
    ng = t * NSTRIPE
    Fs = []
    colv = None
    for x_ref in (x0_ref, x1_ref):
        X3 = x_ref[b]
        thw, ch = X3.shape
        X4 = X3.reshape(ng, 16, ch)
        A = X4[:, 0:8, :] + X4[:, 8:16, :]
        F = jnp.sum(A, axis=1) * (1.0 / 16.0)
        Fs.append(F)
        cv = jnp.sum(F * F, axis=1, keepdims=True)
        colv = cv if colv is None else colv + cv
    S8 = jnp.concatenate([colv[8 * k:8 * (k + 1)] for k in range(t)], axis=1)
    S8 = jnp.maximum(S8, EPS)

    tio = lax.broadcasted_iota(jnp.int32, (NSTRIPE, t), 1)
    m1 = jnp.max(S8, axis=1, keepdims=True)
    i1 = jnp.min(jnp.where(S8 == m1, tio, t), axis=1, keepdims=True)
    Sm = jnp.where(tio == i1, -1.0, S8)
    m2 = jnp.max(Sm, axis=1, keepdims=True)
    i2 = jnp.min(jnp.where(Sm == m2, tio, t), axis=1, keepdims=True)

    i1t = jnp.tile(i1, (t, 1))
    i2t = jnp.tile(i2, (t, 1))
    gio = lax.broadcasted_iota(jnp.int32, (ng, 1), 0)
    tg = gio // NSTRIPE
    Wg = jnp.where((tg == i1t) | (tg == i2t), 0.5, 0.0)

    for j, F in enumerate(Fs):
        ch = F.shape[1]
        FW = F * Wg
        o_ref[b, :, j * ch:(j + 1) * ch] = jnp.sum(
            FW.reshape(t, NSTRIPE, ch), axis=0)


NB = 2


def kernel(x):
    n, c, t, h, w = x.shape
    xt = x.transpose(0, 2, 3, 4, 1).reshape(n, t * h * w, c)
    ch = c // 2
    out = pl.pallas_call(
        _body,
        grid=(n // NB,),
        in_specs=[
            pl.BlockSpec((NB, t * h * w, ch), lambda i: (i, 0, 0)),
            pl.BlockSpec((NB, t * h * w, ch), lambda i: (i, 0, 1)),
        ],
        out_specs=pl.BlockSpec((NB, NSTRIPE, c), lambda i: (i, 0, 0)),
        out_shape=jax.ShapeDtypeStruct((n, NSTRIPE, c), jnp.float32),
        compiler_params=pltpu.CompilerParams(
            dimension_semantics=("parallel",)),
    )(xt, xt)
    return out.reshape(n, NSTRIPE * c)


# speedup vs baseline: 1.1958x; 1.0345x over previous
import jax
import jax.numpy as jnp
from jax import lax
from jax.experimental import pallas as pl
from jax.experimental.pallas import tpu as pltpu

NSTRIPE = 8
EPS = 1e-06


def _body(x0_ref, x1_ref, o_ref):
    for b in range(x0_ref.shape[0]):
        _one(b, x0_ref, x1_ref, o_ref)


def _one(b, x0_ref, x1_ref, o_ref):
    t = 8
    ng = t * NSTRIPE
    Fs = []
    colv = None
    for x_ref in (x0_ref, x1_ref):
        X3 = x_ref[b]
        thw, ch = X3.shape
        X4 = X3.reshape(ng, 16, ch)
        A = X4[:, 0:8, :] + X4[:, 8:16, :]
        F = jnp.sum(A, axis=1)
        Fs.append(F)
        cv = jnp.sum(F * F, axis=1, keepdims=True)
        colv = cv if colv is None else colv + cv
    S8 = jnp.concatenate([colv[8 * k:8 * (k + 1)] for k in range(t)], axis=1)
    S8 = jnp.maximum(S8, 256.0 * EPS)

    tio = lax.broadcasted_iota(jnp.int32, (NSTRIPE, t), 1)
    m1 = jnp.max(S8, axis=1, keepdims=True)
    i1 = jnp.min(jnp.where(S8 == m1, tio, t), axis=1, keepdims=True)
    Sm = jnp.where(tio == i1, -1.0, S8)
    m2 = jnp.max(Sm, axis=1, keepdims=True)
    i2 = jnp.min(jnp.where(Sm == m2, tio, t), axis=1, keepdims=True)

    i1t = jnp.tile(i1, (t, 1))
    i2t = jnp.tile(i2, (t, 1))
    gio = lax.broadcasted_iota(jnp.int32, (ng, 1), 0)
    tg = gio // NSTRIPE
    Wg = jnp.where((tg == i1t) | (tg == i2t), 1.0 / 32.0, 0.0)

    for j, F in enumerate(Fs):
        ch = F.shape[1]
        FW = F * Wg
        o_ref[b, :, j * ch:(j + 1) * ch] = jnp.sum(
            FW.reshape(t, NSTRIPE, ch), axis=0)


NB = 2


def kernel(x):
    n, c, t, h, w = x.shape
    xt = x.transpose(0, 2, 3, 4, 1).reshape(n, t * h * w, c)
    ch = c // 2
    out = pl.pallas_call(
        _body,
        grid=(n // NB,),
        in_specs=[
            pl.BlockSpec((NB, t * h * w, ch), lambda i: (i, 0, 0)),
            pl.BlockSpec((NB, t * h * w, ch), lambda i: (i, 0, 1)),
        ],
        out_specs=pl.BlockSpec((NB, NSTRIPE, c), lambda i: (i, 0, 0)),
        out_shape=jax.ShapeDtypeStruct((n, NSTRIPE, c), jnp.float32),
        compiler_params=pltpu.CompilerParams(
            dimension_semantics=("parallel",)),
    )(xt, xt)
    return out.reshape(n, NSTRIPE * c)


# speedup vs baseline: 1.2464x; 1.0422x over previous
import jax
import jax.numpy as jnp
from jax import lax
from jax.experimental import pallas as pl
from jax.experimental.pallas import tpu as pltpu

NSTRIPE = 8
EPS = 1e-06


def _body(x0_ref, x1_ref, x2_ref, x3_ref, o_ref):
    for b in range(x0_ref.shape[0]):
        _one(b, (x0_ref, x1_ref, x2_ref, x3_ref), o_ref)


def _one(b, x_refs, o_ref):
    t = 8
    ng = t * NSTRIPE
    Fs = []
    colv = None
    for x_ref in x_refs:
        X3 = x_ref[b]
        thw, ch = X3.shape
        X4 = X3.reshape(ng, 16, ch)
        A = X4[:, 0:8, :] + X4[:, 8:16, :]
        F = jnp.sum(A, axis=1)
        Fs.append(F)
        cv = jnp.sum(F * F, axis=1, keepdims=True)
        colv = cv if colv is None else colv + cv
    S8 = jnp.concatenate([colv[8 * k:8 * (k + 1)] for k in range(t)], axis=1)
    S8 = jnp.maximum(S8, 256.0 * EPS)

    tio = lax.broadcasted_iota(jnp.int32, (NSTRIPE, t), 1)
    m1 = jnp.max(S8, axis=1, keepdims=True)
    i1 = jnp.min(jnp.where(S8 == m1, tio, t), axis=1, keepdims=True)
    Sm = jnp.where(tio == i1, -1.0, S8)
    m2 = jnp.max(Sm, axis=1, keepdims=True)
    i2 = jnp.min(jnp.where(Sm == m2, tio, t), axis=1, keepdims=True)

    i1t = jnp.tile(i1, (t, 1))
    i2t = jnp.tile(i2, (t, 1))
    gio = lax.broadcasted_iota(jnp.int32, (ng, 1), 0)
    tg = gio // NSTRIPE
    Wg = jnp.where((tg == i1t) | (tg == i2t), 1.0 / 32.0, 0.0)

    for j, F in enumerate(Fs):
        ch = F.shape[1]
        FW = F * Wg
        o_ref[b, :, j * ch:(j + 1) * ch] = jnp.sum(
            FW.reshape(t, NSTRIPE, ch), axis=0)


NB = 2


def kernel(x):
    n, c, t, h, w = x.shape
    xt = x.transpose(0, 2, 3, 4, 1).reshape(n, t * h * w, c)
    ch = c // 4
    out = pl.pallas_call(
        _body,
        grid=(n // NB,),
        in_specs=[
            pl.BlockSpec((NB, t * h * w, ch), lambda i, j=j: (i, 0, j))
            for j in range(4)
        ],
        out_specs=pl.BlockSpec((NB, NSTRIPE, c), lambda i: (i, 0, 0)),
        out_shape=jax.ShapeDtypeStruct((n, NSTRIPE, c), jnp.float32),
        compiler_params=pltpu.CompilerParams(
            dimension_semantics=("parallel",)),
    )(xt, xt, xt, xt)
    return out.reshape(n, NSTRIPE * c)
